# baseline (speedup 1.0000x reference)
import functools

import jax
import jax.numpy as jnp
from jax.experimental import pallas as pl
from jax.experimental.pallas import tpu as pltpu

BATCH = 4096
IN_DIM = 784
MID = 1024
CODE = 8192
SD = 64
NS = 128
KN = 410
KS = 16

BT_A = 512
BT_B = 512
CT_B = 2048
BT_D = 128


def _enc1_kernel(x_ref, w_ref, b_ref, o_ref):
    o_ref[...] = jnp.maximum(
        jnp.dot(x_ref[...], w_ref[...], preferred_element_type=jnp.float32)
        + b_ref[...], 0.0)


def _enc2_kernel(h1_ref, w2_ref, b2_ref, h2_ref, cs_ref):
    i = pl.program_id(1)
    h2 = jnp.maximum(
        jnp.dot(h1_ref[...], w2_ref[...], preferred_element_type=jnp.float32)
        + b2_ref[...], 0.0)
    h2_ref[...] = h2
    ps = jnp.sum(h2, axis=0, keepdims=True)

    @pl.when(i == 0)
    def _():
        cs_ref[...] = ps

    @pl.when(i != 0)
    def _():
        cs_ref[...] = cs_ref[...] + ps


def _nmask_kernel(cs_ref, m_ref):
    bits = jax.lax.bitcast_convert_type(cs_ref[...], jnp.int32)

    def tstep(k, t):
        cand = t | (1 << (30 - k))
        cnt = jnp.sum((bits >= cand).astype(jnp.int32))
        return jnp.where(cnt >= KN, cand, t)

    t = jax.lax.fori_loop(0, 31, tstep, jnp.int32(0))
    m = jnp.sum((bits > t).astype(jnp.int32))
    r = KN - m
    tie = bits == t
    idx = jax.lax.broadcasted_iota(jnp.int32, (1, CODE), 1)

    def jstep(k, J):
        cand = J | (1 << (13 - k))
        g = jnp.sum((tie & (idx < cand)).astype(jnp.int32))
        return jnp.where((cand <= CODE) & (g <= r), cand, J)

    J = jax.lax.fori_loop(0, 14, jstep, jnp.int32(0))
    m_ref[...] = jnp.where((bits > t) | (tie & (idx < J)), 1.0, 0.0)


def _dec_kernel(h2_ref, nm_ref, w3_ref, b3_ref, w4_ref, b4_ref, o_ref):
    hm = h2_ref[...] * nm_ref[...]
    sn = jax.lax.broadcasted_iota(jnp.int32, (CODE, NS), 0) // SD
    ss_col = jax.lax.broadcasted_iota(jnp.int32, (CODE, NS), 1)
    S = (sn == ss_col).astype(jnp.float32)
    ss = jnp.dot(hm, S, preferred_element_type=jnp.float32,
                 precision=jax.lax.Precision.HIGHEST)

    bits = jax.lax.bitcast_convert_type(ss, jnp.int32)

    def tstep(k, t):
        cand = t | (1 << (30 - k))
        cnt = jnp.sum((bits >= cand).astype(jnp.int32), axis=1, keepdims=True)
        return jnp.where(cnt >= KS, cand, t)

    t = jax.lax.fori_loop(0, 31, tstep, jnp.zeros((BT_D, 1), jnp.int32))
    m = jnp.sum((bits > t).astype(jnp.int32), axis=1, keepdims=True)
    r = KS - m
    tie = bits == t
    sidx = jax.lax.broadcasted_iota(jnp.int32, (1, NS), 1)

    def jstep(k, J):
        cand = J | (1 << (7 - k))
        g = jnp.sum((tie & (sidx < cand)).astype(jnp.int32), axis=1,
                    keepdims=True)
        return jnp.where((cand <= NS) & (g <= r), cand, J)

    J = jax.lax.fori_loop(0, 8, jstep, jnp.zeros((BT_D, 1), jnp.int32))
    smask = ((bits > t) | (tie & (sidx < J))).astype(jnp.float32)

    en = jax.lax.broadcasted_iota(jnp.int32, (NS, CODE), 1) // SD
    es = jax.lax.broadcasted_iota(jnp.int32, (NS, CODE), 0)
    E = (en == es).astype(jnp.float32)
    hs = hm * jnp.dot(smask, E, preferred_element_type=jnp.float32)

    h3 = jnp.maximum(
        jnp.dot(hs.astype(jnp.bfloat16), w3_ref[...],
                preferred_element_type=jnp.float32) + b3_ref[...], 0.0)
    o_ref[...] = jnp.maximum(
        jnp.dot(h3.astype(jnp.bfloat16), w4_ref[...],
                preferred_element_type=jnp.float32) + b4_ref[...], 0.0)


@jax.jit
def kernel(x, W1, b1, W2, b2, W3, b3, W4, b4):
    b1r = b1.reshape(1, MID)
    b2r = b2.reshape(1, CODE)
    b3r = b3.reshape(1, MID)
    b4r = b4.reshape(1, IN_DIM)

    h1 = pl.pallas_call(
        _enc1_kernel,
        grid=(BATCH // BT_A,),
        in_specs=[
            pl.BlockSpec((BT_A, IN_DIM), lambda i: (i, 0)),
            pl.BlockSpec((IN_DIM, MID), lambda i: (0, 0)),
            pl.BlockSpec((1, MID), lambda i: (0, 0)),
        ],
        out_specs=pl.BlockSpec((BT_A, MID), lambda i: (i, 0)),
        out_shape=jax.ShapeDtypeStruct((BATCH, MID), jnp.float32),
    )(x, W1, b1r)

    h2, cs = pl.pallas_call(
        _enc2_kernel,
        grid=(CODE // CT_B, BATCH // BT_B),
        in_specs=[
            pl.BlockSpec((BT_B, MID), lambda j, i: (i, 0)),
            pl.BlockSpec((MID, CT_B), lambda j, i: (0, j)),
            pl.BlockSpec((1, CT_B), lambda j, i: (0, j)),
        ],
        out_specs=[
            pl.BlockSpec((BT_B, CT_B), lambda j, i: (i, j)),
            pl.BlockSpec((1, CT_B), lambda j, i: (0, j)),
        ],
        out_shape=[
            jax.ShapeDtypeStruct((BATCH, CODE), jnp.float32),
            jax.ShapeDtypeStruct((1, CODE), jnp.float32),
        ],
    )(h1, W2, b2r)

    nmask = pl.pallas_call(
        _nmask_kernel,
        out_shape=jax.ShapeDtypeStruct((1, CODE), jnp.float32),
    )(cs)

    out = pl.pallas_call(
        _dec_kernel,
        grid=(BATCH // BT_D,),
        in_specs=[
            pl.BlockSpec((BT_D, CODE), lambda i: (i, 0)),
            pl.BlockSpec((1, CODE), lambda i: (0, 0)),
            pl.BlockSpec((CODE, MID), lambda i: (0, 0)),
            pl.BlockSpec((1, MID), lambda i: (0, 0)),
            pl.BlockSpec((MID, IN_DIM), lambda i: (0, 0)),
            pl.BlockSpec((1, IN_DIM), lambda i: (0, 0)),
        ],
        out_specs=pl.BlockSpec((BT_D, IN_DIM), lambda i: (i, 0)),
        out_shape=jax.ShapeDtypeStruct((BATCH, IN_DIM), jnp.float32),
        compiler_params=pltpu.CompilerParams(
            vmem_limit_bytes=100 * 1024 * 1024,
        ),
    )(h2, nmask, W3.astype(jnp.bfloat16), b3r, W4.astype(jnp.bfloat16), b4r)
    return out


# speedup vs baseline: 1.1149x; 1.1149x over previous
import functools

import jax
import jax.numpy as jnp
from jax.experimental import pallas as pl
from jax.experimental.pallas import tpu as pltpu

BATCH = 4096
IN_DIM = 784
MID = 1024
CODE = 8192
SD = 64
NS = 128
KN = 410
KS = 16

BT_A = 512
BT_B = 512
CT_B = 2048
BT_D = 128


def _enc1_kernel(x_ref, w_ref, b_ref, o_ref):
    o_ref[...] = jnp.maximum(
        jnp.dot(x_ref[...], w_ref[...], preferred_element_type=jnp.float32)
        + b_ref[...], 0.0)


def _enc2_kernel(h1_ref, w2_ref, b2_ref, h2_ref, cs_ref):
    i = pl.program_id(1)
    h2 = jnp.maximum(
        jnp.dot(h1_ref[...], w2_ref[...], preferred_element_type=jnp.float32)
        + b2_ref[...], 0.0)
    h2_ref[...] = h2
    ps = jnp.sum(h2, axis=0, keepdims=True)

    @pl.when(i == 0)
    def _():
        cs_ref[...] = ps

    @pl.when(i != 0)
    def _():
        cs_ref[...] = cs_ref[...] + ps


def _nmask_kernel(cs_ref, m_ref):
    bits = jax.lax.bitcast_convert_type(cs_ref[...], jnp.int32)

    def tstep(k, t):
        cand = t | (1 << (30 - k))
        cnt = jnp.sum((bits >= cand).astype(jnp.int32))
        return jnp.where(cnt >= KN, cand, t)

    t = jax.lax.fori_loop(0, 31, tstep, jnp.int32(0))
    m = jnp.sum((bits > t).astype(jnp.int32))
    r = KN - m
    tie = bits == t
    idx = jax.lax.broadcasted_iota(jnp.int32, (1, CODE), 1)

    def jstep(k, J):
        cand = J | (1 << (13 - k))
        g = jnp.sum((tie & (idx < cand)).astype(jnp.int32))
        return jnp.where((cand <= CODE) & (g <= r), cand, J)

    J = jax.lax.fori_loop(0, 14, jstep, jnp.int32(0))
    m_ref[...] = jnp.where((bits > t) | (tie & (idx < J)), 1.0, 0.0)


def _dec_kernel(h2_ref, nm_ref, w3_ref, b3_ref, w4_ref, b4_ref, o_ref):
    hm = h2_ref[...] * nm_ref[...]
    sn = jax.lax.broadcasted_iota(jnp.int32, (CODE, NS), 0) // SD
    ss_col = jax.lax.broadcasted_iota(jnp.int32, (CODE, NS), 1)
    S = (sn == ss_col).astype(jnp.bfloat16)
    hm_hi = hm.astype(jnp.bfloat16)
    hm_lo = (hm - hm_hi.astype(jnp.float32)).astype(jnp.bfloat16)
    ss = (jnp.dot(hm_hi, S, preferred_element_type=jnp.float32)
          + jnp.dot(hm_lo, S, preferred_element_type=jnp.float32))

    bits = jax.lax.bitcast_convert_type(ss, jnp.int32)

    def tstep(k, t):
        cand = t | (1 << (30 - k))
        cnt = jnp.sum((bits >= cand).astype(jnp.int32), axis=1, keepdims=True)
        return jnp.where(cnt >= KS, cand, t)

    t = jax.lax.fori_loop(0, 31, tstep, jnp.zeros((BT_D, 1), jnp.int32))
    m = jnp.sum((bits > t).astype(jnp.int32), axis=1, keepdims=True)
    r = KS - m
    tie = bits == t
    sidx = jax.lax.broadcasted_iota(jnp.int32, (1, NS), 1)

    def jstep(k, J):
        cand = J | (1 << (7 - k))
        g = jnp.sum((tie & (sidx < cand)).astype(jnp.int32), axis=1,
                    keepdims=True)
        return jnp.where((cand <= NS) & (g <= r), cand, J)

    J = jax.lax.fori_loop(0, 8, jstep, jnp.zeros((BT_D, 1), jnp.int32))
    smask = ((bits > t) | (tie & (sidx < J))).astype(jnp.float32)

    en = jax.lax.broadcasted_iota(jnp.int32, (NS, CODE), 1) // SD
    es = jax.lax.broadcasted_iota(jnp.int32, (NS, CODE), 0)
    E = (en == es).astype(jnp.float32)
    hs = hm * jnp.dot(smask, E, preferred_element_type=jnp.float32)

    h3 = jnp.maximum(
        jnp.dot(hs.astype(jnp.bfloat16), w3_ref[...],
                preferred_element_type=jnp.float32) + b3_ref[...], 0.0)
    o_ref[...] = jnp.maximum(
        jnp.dot(h3.astype(jnp.bfloat16), w4_ref[...],
                preferred_element_type=jnp.float32) + b4_ref[...], 0.0)


@jax.jit
def kernel(x, W1, b1, W2, b2, W3, b3, W4, b4):
    b1r = b1.reshape(1, MID)
    b2r = b2.reshape(1, CODE)
    b3r = b3.reshape(1, MID)
    b4r = b4.reshape(1, IN_DIM)

    h1 = pl.pallas_call(
        _enc1_kernel,
        grid=(BATCH // BT_A,),
        in_specs=[
            pl.BlockSpec((BT_A, IN_DIM), lambda i: (i, 0)),
            pl.BlockSpec((IN_DIM, MID), lambda i: (0, 0)),
            pl.BlockSpec((1, MID), lambda i: (0, 0)),
        ],
        out_specs=pl.BlockSpec((BT_A, MID), lambda i: (i, 0)),
        out_shape=jax.ShapeDtypeStruct((BATCH, MID), jnp.float32),
    )(x, W1, b1r)

    h2, cs = pl.pallas_call(
        _enc2_kernel,
        grid=(CODE // CT_B, BATCH // BT_B),
        in_specs=[
            pl.BlockSpec((BT_B, MID), lambda j, i: (i, 0)),
            pl.BlockSpec((MID, CT_B), lambda j, i: (0, j)),
            pl.BlockSpec((1, CT_B), lambda j, i: (0, j)),
        ],
        out_specs=[
            pl.BlockSpec((BT_B, CT_B), lambda j, i: (i, j)),
            pl.BlockSpec((1, CT_B), lambda j, i: (0, j)),
        ],
        out_shape=[
            jax.ShapeDtypeStruct((BATCH, CODE), jnp.float32),
            jax.ShapeDtypeStruct((1, CODE), jnp.float32),
        ],
    )(h1, W2, b2r)

    nmask = pl.pallas_call(
        _nmask_kernel,
        out_shape=jax.ShapeDtypeStruct((1, CODE), jnp.float32),
    )(cs)

    out = pl.pallas_call(
        _dec_kernel,
        grid=(BATCH // BT_D,),
        in_specs=[
            pl.BlockSpec((BT_D, CODE), lambda i: (i, 0)),
            pl.BlockSpec((1, CODE), lambda i: (0, 0)),
            pl.BlockSpec((CODE, MID), lambda i: (0, 0)),
            pl.BlockSpec((1, MID), lambda i: (0, 0)),
            pl.BlockSpec((MID, IN_DIM), lambda i: (0, 0)),
            pl.BlockSpec((1, IN_DIM), lambda i: (0, 0)),
        ],
        out_specs=pl.BlockSpec((BT_D, IN_DIM), lambda i: (i, 0)),
        out_shape=jax.ShapeDtypeStruct((BATCH, IN_DIM), jnp.float32),
        compiler_params=pltpu.CompilerParams(
            vmem_limit_bytes=100 * 1024 * 1024,
        ),
    )(h2, nmask, W3.astype(jnp.bfloat16), b3r, W4.astype(jnp.bfloat16), b4r)
    return out


# speedup vs baseline: 1.4166x; 1.2706x over previous
import jax
import jax.numpy as jnp
from jax.experimental import pallas as pl
from jax.experimental.pallas import tpu as pltpu

BATCH = 4096
IN_DIM = 784
MID = 1024
CODE = 8192
SD = 64
NS = 128
KN = 410
KS = 16
CC = 512

BT_A = 512
BT_B = 512
CT_B = 2048
KB = 1024
BT_D = 256


def _enc1_kernel(x_ref, w_ref, b_ref, o_ref):
    o_ref[...] = jnp.maximum(
        jnp.dot(x_ref[...], w_ref[...], preferred_element_type=jnp.float32)
        + b_ref[...], 0.0)


def _enc2_kernel(h1_ref, w2_ref, b2_ref, h2_ref, cs_ref):
    i = pl.program_id(1)
    h2 = jnp.maximum(
        jnp.dot(h1_ref[...], w2_ref[...], preferred_element_type=jnp.float32)
        + b2_ref[...], 0.0)
    h2_ref[...] = h2
    ps = jnp.sum(h2, axis=0, keepdims=True)

    @pl.when(i == 0)
    def _():
        cs_ref[...] = ps

    @pl.when(i != 0)
    def _():
        cs_ref[...] = cs_ref[...] + ps


def _nmask_kernel(cs_ref, m_ref, slot_ref):
    bits = jax.lax.bitcast_convert_type(cs_ref[...], jnp.int32)

    def tstep(k, t):
        cand = t | (1 << (30 - k))
        cnt = jnp.sum((bits >= cand).astype(jnp.int32))
        return jnp.where(cnt >= KN, cand, t)

    t = jax.lax.fori_loop(0, 31, tstep, jnp.int32(0))
    m = jnp.sum((bits > t).astype(jnp.int32))
    r = KN - m
    tie = bits == t
    idx = jax.lax.broadcasted_iota(jnp.int32, (1, CODE), 1)

    def jstep(k, J):
        cand = J | (1 << (13 - k))
        g = jnp.sum((tie & (idx < cand)).astype(jnp.int32))
        return jnp.where((cand <= CODE) & (g <= r), cand, J)

    J = jax.lax.fori_loop(0, 14, jstep, jnp.int32(0))
    mask = (bits > t) | (tie & (idx < J))
    m_ref[...] = jnp.where(mask, 1.0, 0.0)
    x = m_ref[...]
    sh = 1
    while sh < CODE:
        x = x + jnp.concatenate(
            [jnp.zeros((1, sh), jnp.float32), x[:, :CODE - sh]], axis=1)
        sh *= 2
    slot_ref[...] = jnp.where(mask, x - 1.0, -1.0)


def _build_kernel(slot_ref, w3_ref, pt_ref, w3c_ref, st_ref, sel_ref):
    k = pl.program_id(0)
    slot_blk = slot_ref[...]
    jrow = jax.lax.broadcasted_iota(jnp.int32, (CC, KB), 0)
    pt_blk = (jrow == slot_blk.astype(jnp.int32)).astype(jnp.float32)
    pt_blk = jnp.where(slot_blk >= 0.0, pt_blk, 0.0)
    pt_ref[...] = pt_blk.astype(jnp.bfloat16)
    stripe = ((jax.lax.broadcasted_iota(jnp.int32, (1, KB), 1)
               + k * KB) // SD).astype(jnp.bfloat16)
    st_part = jax.lax.dot_general(
        stripe, pt_blk.astype(jnp.bfloat16), (((1,), (1,)), ((), ())),
        preferred_element_type=jnp.float32)
    w3c_part = jnp.dot(pt_blk.astype(jnp.bfloat16),
                       w3_ref[...].astype(jnp.bfloat16),
                       preferred_element_type=jnp.float32)

    @pl.when(k == 0)
    def _():
        st_ref[...] = st_part
        w3c_ref[...] = w3c_part

    @pl.when(k != 0)
    def _():
        st_ref[...] = st_ref[...] + st_part
        w3c_ref[...] = w3c_ref[...] + w3c_part

    @pl.when(k == CODE // KB - 1)
    def _():
        srow = jax.lax.broadcasted_iota(jnp.int32, (NS, CC), 0)
        sel_ref[...] = (srow == st_ref[...].astype(jnp.int32)).astype(
            jnp.bfloat16)


def _dec_kernel(h2_ref, nm_ref, pt_ref, w3c_ref, sel_ref, b3_ref, w4_ref,
                b4_ref, o_ref):
    hm = h2_ref[...] * nm_ref[...]
    hm_hi = hm.astype(jnp.bfloat16)
    hm_lo = (hm - hm_hi.astype(jnp.float32)).astype(jnp.bfloat16)
    sn = jax.lax.broadcasted_iota(jnp.int32, (CODE, NS), 0) // SD
    sc = jax.lax.broadcasted_iota(jnp.int32, (CODE, NS), 1)
    S = (sn == sc).astype(jnp.bfloat16)
    ss = (jnp.dot(hm_hi, S, preferred_element_type=jnp.float32)
          + jnp.dot(hm_lo, S, preferred_element_type=jnp.float32))

    bits = jax.lax.bitcast_convert_type(ss, jnp.int32)

    def tstep(k, t):
        cand = t | (1 << (30 - k))
        cnt = jnp.sum((bits >= cand).astype(jnp.int32), axis=1, keepdims=True)
        return jnp.where(cnt >= KS, cand, t)

    t = jax.lax.fori_loop(0, 31, tstep, jnp.zeros((BT_D, 1), jnp.int32))
    m = jnp.sum((bits > t).astype(jnp.int32), axis=1, keepdims=True)
    r = KS - m
    tie = bits == t
    sidx = jax.lax.broadcasted_iota(jnp.int32, (1, NS), 1)

    def jstep(k, J):
        cand = J | (1 << (7 - k))
        g = jnp.sum((tie & (sidx < cand)).astype(jnp.int32), axis=1,
                    keepdims=True)
        return jnp.where((cand <= NS) & (g <= r), cand, J)

    J = jax.lax.fori_loop(0, 8, jstep, jnp.zeros((BT_D, 1), jnp.int32))
    smask = ((bits > t) | (tie & (sidx < J))).astype(jnp.bfloat16)

    c = jax.lax.dot_general(hm_hi, pt_ref[...], (((1,), (1,)), ((), ())),
                            preferred_element_type=jnp.float32)
    smask_c = jnp.dot(smask, sel_ref[...],
                      preferred_element_type=jnp.float32)
    cm = (c * smask_c).astype(jnp.bfloat16)
    h3 = jnp.maximum(
        jnp.dot(cm, w3c_ref[...].astype(jnp.bfloat16),
                preferred_element_type=jnp.float32) + b3_ref[...], 0.0)
    o_ref[...] = jnp.maximum(
        jnp.dot(h3.astype(jnp.bfloat16), w4_ref[...],
                preferred_element_type=jnp.float32) + b4_ref[...], 0.0)


@jax.jit
def kernel(x, W1, b1, W2, b2, W3, b3, W4, b4):
    b1r = b1.reshape(1, MID)
    b2r = b2.reshape(1, CODE)
    b3r = b3.reshape(1, MID)
    b4r = b4.reshape(1, IN_DIM)

    h1 = pl.pallas_call(
        _enc1_kernel,
        grid=(BATCH // BT_A,),
        in_specs=[
            pl.BlockSpec((BT_A, IN_DIM), lambda i: (i, 0)),
            pl.BlockSpec((IN_DIM, MID), lambda i: (0, 0)),
            pl.BlockSpec((1, MID), lambda i: (0, 0)),
        ],
        out_specs=pl.BlockSpec((BT_A, MID), lambda i: (i, 0)),
        out_shape=jax.ShapeDtypeStruct((BATCH, MID), jnp.float32),
    )(x, W1, b1r)

    h2, cs = pl.pallas_call(
        _enc2_kernel,
        grid=(CODE // CT_B, BATCH // BT_B),
        in_specs=[
            pl.BlockSpec((BT_B, MID), lambda j, i: (i, 0)),
            pl.BlockSpec((MID, CT_B), lambda j, i: (0, j)),
            pl.BlockSpec((1, CT_B), lambda j, i: (0, j)),
        ],
        out_specs=[
            pl.BlockSpec((BT_B, CT_B), lambda j, i: (i, j)),
            pl.BlockSpec((1, CT_B), lambda j, i: (0, j)),
        ],
        out_shape=[
            jax.ShapeDtypeStruct((BATCH, CODE), jnp.float32),
            jax.ShapeDtypeStruct((1, CODE), jnp.float32),
        ],
    )(h1, W2, b2r)

    nmask, slotm = pl.pallas_call(
        _nmask_kernel,
        out_shape=[
            jax.ShapeDtypeStruct((1, CODE), jnp.float32),
            jax.ShapeDtypeStruct((1, CODE), jnp.float32),
        ],
    )(cs)

    pt, w3c, st, sel = pl.pallas_call(
        _build_kernel,
        grid=(CODE // KB,),
        in_specs=[
            pl.BlockSpec((1, KB), lambda k: (0, k)),
            pl.BlockSpec((KB, MID), lambda k: (k, 0)),
        ],
        out_specs=[
            pl.BlockSpec((CC, KB), lambda k: (0, k)),
            pl.BlockSpec((CC, MID), lambda k: (0, 0)),
            pl.BlockSpec((1, CC), lambda k: (0, 0)),
            pl.BlockSpec((NS, CC), lambda k: (0, 0)),
        ],
        out_shape=[
            jax.ShapeDtypeStruct((CC, CODE), jnp.bfloat16),
            jax.ShapeDtypeStruct((CC, MID), jnp.float32),
            jax.ShapeDtypeStruct((1, CC), jnp.float32),
            jax.ShapeDtypeStruct((NS, CC), jnp.bfloat16),
        ],
    )(slotm, W3)

    out = pl.pallas_call(
        _dec_kernel,
        grid=(BATCH // BT_D,),
        in_specs=[
            pl.BlockSpec((BT_D, CODE), lambda i: (i, 0)),
            pl.BlockSpec((1, CODE), lambda i: (0, 0)),
            pl.BlockSpec((CC, CODE), lambda i: (0, 0)),
            pl.BlockSpec((CC, MID), lambda i: (0, 0)),
            pl.BlockSpec((NS, CC), lambda i: (0, 0)),
            pl.BlockSpec((1, MID), lambda i: (0, 0)),
            pl.BlockSpec((MID, IN_DIM), lambda i: (0, 0)),
            pl.BlockSpec((1, IN_DIM), lambda i: (0, 0)),
        ],
        out_specs=pl.BlockSpec((BT_D, IN_DIM), lambda i: (i, 0)),
        out_shape=jax.ShapeDtypeStruct((BATCH, IN_DIM), jnp.float32),
        compiler_params=pltpu.CompilerParams(
            vmem_limit_bytes=100 * 1024 * 1024,
        ),
    )(h2, nmask, pt, w3c, sel, b3r, W4.astype(jnp.bfloat16), b4r)
    return out
